# baseline (speedup 1.0000x reference)
import functools

import jax
import jax.numpy as jnp
from jax import lax
from jax.experimental import pallas as pl
from jax.experimental.pallas import tpu as pltpu
from jax.experimental.pallas import tpu_sc as plsc

_NC = 2
_NS = 16
_NW = _NC * _NS
_L = 16
_NB = 16


def kernel(x, hex_idx):
    n_fine, feat = x.shape
    num_nodes = (n_fine + 6) // 4
    k = hex_idx.shape[1]
    idx_blk = _NB * k
    row_blk = idx_blk * feat
    out_blk = _NB * feat

    n_main = (num_nodes // _NB) * _NB
    n_tail = num_nodes - n_main
    bpw = (n_main // _NB) // _NW
    assert (n_main // _NB) % _NW == 0 and bpw % 2 == 0

    hx = hex_idx[:num_nodes].reshape(-1)

    mesh = plsc.VectorSubcoreMesh(
        core_axis_name="c", subcore_axis_name="s",
        num_cores=_NC, num_subcores=_NS,
    )

    @functools.partial(
        pl.kernel,
        out_type=jax.ShapeDtypeStruct((num_nodes * feat,), jnp.float32),
        mesh=mesh,
        scratch_types=[
            pltpu.VMEM((bpw * idx_blk,), jnp.int32),
            pltpu.VMEM((idx_blk, feat), jnp.float32),
            pltpu.VMEM((idx_blk, feat), jnp.float32),
            pltpu.VMEM((out_blk,), jnp.float32),
            pltpu.VMEM((out_blk,), jnp.float32),
            pltpu.SemaphoreType.DMA,
            pltpu.SemaphoreType.DMA,
            pltpu.SemaphoreType.DMA,
            pltpu.SemaphoreType.DMA,
        ],
        compiler_params=pltpu.CompilerParams(
            use_tc_tiling_on_sc=False, needs_layout_passes=False),
    )
    def body(x_hbm, hx_hbm, out_hbm, idx_all, rows0, rows1, out0, out1,
             gsem0, gsem1, ssem0, ssem1):
        wid = lax.axis_index("s") * _NC + lax.axis_index("c")
        lane = lax.iota(jnp.int32, _L)
        blk0 = wid * bpw

        def g_copy(b, rows, gsem):
            return pltpu.make_async_copy(
                x_hbm.at[idx_all.at[pl.ds(b * idx_blk, idx_blk)]],
                rows, gsem)

        def s_copy(b, out_v, ssem):
            return pltpu.make_async_copy(
                out_v, out_hbm.at[pl.ds((blk0 + b) * out_blk, out_blk)], ssem)

        klane = k * lane

        def compute(rows, out_v, n_iter=_NB * _L):
            def step(i, _):
                p0 = (k * _L) * i + klane
                acc = None
                for t in range(k):
                    p = p0 + t
                    v = plsc.load_gather(
                        rows,
                        [lax.shift_right_logical(p, 8),
                         lax.bitwise_and(p, 255)])
                    acc = v if acc is None else jnp.maximum(acc, v)
                out_v[pl.ds(i * _L, _L)] = acc
                return _
            lax.fori_loop(0, n_iter, step, 0, unroll=4)

        pltpu.sync_copy(hx_hbm.at[pl.ds(blk0 * idx_blk, bpw * idx_blk)],
                        idx_all)
        g_copy(0, rows0, gsem0).start()

        def half(b, rows, out_v, gsem, ssem, m):
            g_copy(b, rows, gsem).wait()

            @pl.when(m >= 1)
            def _():
                s_copy(b, out_v, ssem).wait()
            compute(rows, out_v)
            s_copy(b, out_v, ssem).start()

        def pair(m, carry):
            b0 = 2 * m
            g_copy(b0 + 1, rows1, gsem1).start()
            half(b0, rows0, out0, gsem0, ssem0, m)

            @pl.when(m < bpw // 2 - 1)
            def _():
                g_copy(b0 + 2, rows0, gsem0).start()
            half(b0 + 1, rows1, out1, gsem1, ssem1, m)
            return carry

        lax.fori_loop(0, bpw // 2, pair, 0)
        s_copy(bpw - 2, out0, ssem0).wait()
        s_copy(bpw - 1, out1, ssem1).wait()

        if n_tail:
            @pl.when(wid == 0)
            def _tail():
                tidx = n_tail * k
                pltpu.sync_copy(hx_hbm.at[pl.ds(n_main * k, tidx)],
                                idx_all.at[pl.ds(0, tidx)])
                iv = idx_all[pl.ds(0, _L)]
                idx_all[pl.ds(0, _L)] = jnp.where(lane < tidx, iv, 0)
                pltpu.async_copy(
                    x_hbm.at[idx_all.at[pl.ds(0, _L)]],
                    rows0.at[pl.ds(0, _L)], gsem0).wait()
                compute(rows0, out0, n_iter=n_tail * _L)
                pltpu.sync_copy(out0.at[pl.ds(0, n_tail * feat)],
                                out_hbm.at[pl.ds(n_main * feat, n_tail * feat)])

    out = body(x, hx)
    return out.reshape(num_nodes, feat)


# speedup vs baseline: 1.2334x; 1.2334x over previous
import functools

import jax
import jax.numpy as jnp
from jax import lax
from jax.experimental import pallas as pl
from jax.experimental.pallas import tpu as pltpu
from jax.experimental.pallas import tpu_sc as plsc

_NC = 2
_NS = 16
_NW = _NC * _NS
_L = 16
_NB = 16


def kernel(x, hex_idx):
    n_fine, feat = x.shape
    num_nodes = (n_fine + 6) // 4
    k = hex_idx.shape[1]
    idx_blk = _NB * k
    row_blk = idx_blk * feat
    out_blk = _NB * feat

    n_main = (num_nodes // _NB) * _NB
    n_tail = num_nodes - n_main
    bpw = (n_main // _NB) // _NW
    assert (n_main // _NB) % _NW == 0 and bpw % 2 == 0

    hx = hex_idx[:num_nodes].reshape(-1)

    mesh = plsc.VectorSubcoreMesh(
        core_axis_name="c", subcore_axis_name="s",
        num_cores=_NC, num_subcores=_NS,
    )

    @functools.partial(
        pl.kernel,
        out_type=jax.ShapeDtypeStruct((num_nodes * feat,), jnp.float32),
        mesh=mesh,
        scratch_types=[
            pltpu.VMEM((bpw * idx_blk,), jnp.int32),
            pltpu.VMEM((idx_blk, feat), jnp.float32),
            pltpu.VMEM((idx_blk, feat), jnp.float32),
            pltpu.VMEM((out_blk,), jnp.float32),
            pltpu.VMEM((out_blk,), jnp.float32),
            pltpu.SemaphoreType.DMA,
            pltpu.SemaphoreType.DMA,
            pltpu.SemaphoreType.DMA,
            pltpu.SemaphoreType.DMA,
        ],
        compiler_params=pltpu.CompilerParams(
            use_tc_tiling_on_sc=True, needs_layout_passes=False),
    )
    def body(x_hbm, hx_hbm, out_hbm, idx_all, rows0, rows1, out0, out1,
             gsem0, gsem1, ssem0, ssem1):
        wid = lax.axis_index("s") * _NC + lax.axis_index("c")
        lane = lax.iota(jnp.int32, _L)
        blk0 = wid * bpw

        def g_copy(b, rows, gsem):
            return pltpu.make_async_copy(
                x_hbm.at[idx_all.at[pl.ds(b * idx_blk, idx_blk)]],
                rows, gsem)

        def s_copy(b, out_v, ssem):
            return pltpu.make_async_copy(
                out_v, out_hbm.at[pl.ds((blk0 + b) * out_blk, out_blk)], ssem)

        klane = k * lane

        def compute(rows, out_v, n_iter=_NB * _L):
            def step(i, _):
                p0 = (k * _L) * i + klane
                acc = None
                for t in range(k):
                    p = p0 + t
                    v = plsc.load_gather(
                        rows,
                        [lax.shift_right_logical(p, 8),
                         lax.bitwise_and(p, 255)])
                    acc = v if acc is None else jnp.maximum(acc, v)
                out_v[pl.ds(i * _L, _L)] = acc
                return _
            lax.fori_loop(0, n_iter, step, 0, unroll=4)

        pltpu.sync_copy(hx_hbm.at[pl.ds(blk0 * idx_blk, bpw * idx_blk)],
                        idx_all)
        g_copy(0, rows0, gsem0).start()

        def half(b, rows, out_v, gsem, ssem, m):
            g_copy(b, rows, gsem).wait()

            @pl.when(m >= 1)
            def _():
                s_copy(b, out_v, ssem).wait()
            compute(rows, out_v)
            s_copy(b, out_v, ssem).start()

        def pair(m, carry):
            b0 = 2 * m
            g_copy(b0 + 1, rows1, gsem1).start()
            half(b0, rows0, out0, gsem0, ssem0, m)

            @pl.when(m < bpw // 2 - 1)
            def _():
                g_copy(b0 + 2, rows0, gsem0).start()
            half(b0 + 1, rows1, out1, gsem1, ssem1, m)
            return carry

        lax.fori_loop(0, bpw // 2, pair, 0)
        s_copy(bpw - 2, out0, ssem0).wait()
        s_copy(bpw - 1, out1, ssem1).wait()

        if n_tail:
            @pl.when(wid == 0)
            def _tail():
                tidx = n_tail * k
                pltpu.sync_copy(hx_hbm.at[pl.ds(n_main * k, tidx)],
                                idx_all.at[pl.ds(0, tidx)])
                iv = idx_all[pl.ds(0, _L)]
                idx_all[pl.ds(0, _L)] = jnp.where(lane < tidx, iv, 0)
                pltpu.async_copy(
                    x_hbm.at[idx_all.at[pl.ds(0, _L)]],
                    rows0.at[pl.ds(0, _L)], gsem0).wait()
                compute(rows0, out0, n_iter=n_tail * _L)
                pltpu.sync_copy(out0.at[pl.ds(0, n_tail * feat)],
                                out_hbm.at[pl.ds(n_main * feat, n_tail * feat)])

    out = body(x, hx)
    return out.reshape(num_nodes, feat)


# speedup vs baseline: 1.2793x; 1.0372x over previous
import functools

import jax
import jax.numpy as jnp
from jax import lax
from jax.experimental import pallas as pl
from jax.experimental.pallas import tpu as pltpu
from jax.experimental.pallas import tpu_sc as plsc

_NC = 2
_NS = 16
_NW = _NC * _NS
_L = 16
_NB = 16


def kernel(x, hex_idx):
    n_fine, feat = x.shape
    num_nodes = (n_fine + 6) // 4
    k = hex_idx.shape[1]
    idx_blk = _NB * k
    row_blk = idx_blk * feat
    out_blk = _NB * feat

    n_main = (num_nodes // _NB) * _NB
    n_tail = num_nodes - n_main
    bpw = (n_main // _NB) // _NW
    assert (n_main // _NB) % _NW == 0 and bpw % 2 == 0

    hx = hex_idx.reshape(-1)

    mesh = plsc.VectorSubcoreMesh(
        core_axis_name="c", subcore_axis_name="s",
        num_cores=_NC, num_subcores=_NS,
    )

    @functools.partial(
        pl.kernel,
        out_type=jax.ShapeDtypeStruct((num_nodes, feat), jnp.float32),
        mesh=mesh,
        scratch_types=[
            pltpu.VMEM((bpw * idx_blk,), jnp.int32),
            pltpu.VMEM((idx_blk, feat), jnp.float32),
            pltpu.VMEM((idx_blk, feat), jnp.float32),
            pltpu.VMEM((_NB, feat), jnp.float32),
            pltpu.VMEM((_NB, feat), jnp.float32),
            pltpu.SemaphoreType.DMA,
            pltpu.SemaphoreType.DMA,
            pltpu.SemaphoreType.DMA,
            pltpu.SemaphoreType.DMA,
        ],
        compiler_params=pltpu.CompilerParams(
            use_tc_tiling_on_sc=True, needs_layout_passes=False),
    )
    def body(x_hbm, hx_hbm, out_hbm, idx_all, rows0, rows1, out0, out1,
             gsem0, gsem1, ssem0, ssem1):
        wid = lax.axis_index("s") * _NC + lax.axis_index("c")
        lane = lax.iota(jnp.int32, _L)
        blk0 = wid * bpw

        def g_copy(b, rows, gsem):
            return pltpu.make_async_copy(
                x_hbm.at[idx_all.at[pl.ds(b * idx_blk, idx_blk)]],
                rows, gsem)

        def s_copy(b, out_v, ssem):
            return pltpu.make_async_copy(
                out_v, out_hbm.at[pl.ds((blk0 + b) * _NB, _NB)], ssem)

        klane = k * lane

        def compute(rows, out_v, n_iter=_NB * _L):
            def step(i, _):
                p0 = (k * _L) * i + klane
                acc = None
                for t in range(k):
                    p = p0 + t
                    v = plsc.load_gather(
                        rows,
                        [lax.shift_right_logical(p, 8),
                         lax.bitwise_and(p, 255)])
                    acc = v if acc is None else jnp.maximum(acc, v)
                out_v[lax.shift_right_logical(i, 4),
                      pl.ds((i * _L) & (feat - 1), _L)] = acc
                return _
            lax.fori_loop(0, n_iter, step, 0, unroll=4)

        pltpu.sync_copy(hx_hbm.at[pl.ds(blk0 * idx_blk, bpw * idx_blk)],
                        idx_all)
        g_copy(0, rows0, gsem0).start()

        def half(b, rows, out_v, gsem, ssem, m):
            g_copy(b, rows, gsem).wait()

            @pl.when(m >= 1)
            def _():
                s_copy(b, out_v, ssem).wait()
            compute(rows, out_v)
            s_copy(b, out_v, ssem).start()

        def pair(m, carry):
            b0 = 2 * m
            g_copy(b0 + 1, rows1, gsem1).start()
            half(b0, rows0, out0, gsem0, ssem0, m)

            @pl.when(m < bpw // 2 - 1)
            def _():
                g_copy(b0 + 2, rows0, gsem0).start()
            half(b0 + 1, rows1, out1, gsem1, ssem1, m)
            return carry

        lax.fori_loop(0, bpw // 2, pair, 0)
        s_copy(bpw - 2, out0, ssem0).wait()
        s_copy(bpw - 1, out1, ssem1).wait()

        if n_tail:
            @pl.when(wid == 0)
            def _tail():
                tidx = n_tail * k
                pltpu.sync_copy(hx_hbm.at[pl.ds(n_main * k, tidx)],
                                idx_all.at[pl.ds(0, tidx)])
                iv = idx_all[pl.ds(0, _L)]
                idx_all[pl.ds(0, _L)] = jnp.where(lane < tidx, iv, 0)
                pltpu.async_copy(
                    x_hbm.at[idx_all.at[pl.ds(0, _L)]],
                    rows0.at[pl.ds(0, _L)], gsem0).wait()
                compute(rows0, out0, n_iter=n_tail * _L)
                pltpu.sync_copy(out0.at[pl.ds(0, n_tail)],
                                out_hbm.at[pl.ds(n_main, n_tail)])

    return body(x, hx)


# speedup vs baseline: 1.3121x; 1.0256x over previous
import functools

import jax
import jax.numpy as jnp
from jax import lax
from jax.experimental import pallas as pl
from jax.experimental.pallas import tpu as pltpu
from jax.experimental.pallas import tpu_sc as plsc

_NC = 2
_NS = 16
_NW = _NC * _NS
_L = 16
_NB = 16


def kernel(x, hex_idx):
    n_fine, feat = x.shape
    num_nodes = (n_fine + 6) // 4
    k = hex_idx.shape[1]
    idx_blk = _NB * k
    row_blk = idx_blk * feat
    out_blk = _NB * feat

    n_main = (num_nodes // _NB) * _NB
    n_tail = num_nodes - n_main
    bpw = (n_main // _NB) // _NW
    assert (n_main // _NB) % _NW == 0 and bpw % 2 == 0

    hx = hex_idx.reshape(-1)

    mesh = plsc.VectorSubcoreMesh(
        core_axis_name="c", subcore_axis_name="s",
        num_cores=_NC, num_subcores=_NS,
    )

    @functools.partial(
        pl.kernel,
        out_type=jax.ShapeDtypeStruct((num_nodes, feat), jnp.float32),
        mesh=mesh,
        scratch_types=[
            pltpu.VMEM((bpw * idx_blk,), jnp.int32),
            pltpu.VMEM((idx_blk, feat), jnp.float32),
            pltpu.VMEM((idx_blk, feat), jnp.float32),
            pltpu.VMEM((_NB, feat), jnp.float32),
            pltpu.VMEM((_NB, feat), jnp.float32),
            pltpu.SemaphoreType.DMA,
            pltpu.SemaphoreType.DMA,
            pltpu.SemaphoreType.DMA,
            pltpu.SemaphoreType.DMA,
        ],
        compiler_params=pltpu.CompilerParams(
            use_tc_tiling_on_sc=True, needs_layout_passes=False),
    )
    def body(x_hbm, hx_hbm, out_hbm, idx_all, rows0, rows1, out0, out1,
             gsem0, gsem1, ssem0, ssem1):
        wid = lax.axis_index("s") * _NC + lax.axis_index("c")
        lane = lax.iota(jnp.int32, _L)
        blk0 = wid * bpw

        def g_copy(b, rows, gsem):
            return pltpu.make_async_copy(
                x_hbm.at[idx_all.at[pl.ds(b * idx_blk, idx_blk)]],
                rows, gsem)

        def s_copy(b, out_v, ssem):
            return pltpu.make_async_copy(
                out_v, out_hbm.at[pl.ds((blk0 + b) * _NB, _NB)], ssem)

        klane = k * lane

        def compute(rows, out_v, nodes=_NB):
            def outer(v, carry):
                p0 = (k * _L) * v + klane
                rc = [(lax.shift_right_logical(p0 + t, 8),
                       lax.bitwise_and(p0 + t, 255)) for t in range(k)]

                def inner(n, c2):
                    kn = k * n
                    acc = None
                    for r0, c0 in rc:
                        vv = plsc.load_gather(rows, [r0 + kn, c0])
                        acc = vv if acc is None else jnp.maximum(acc, vv)
                    out_v[n, pl.ds(v * _L, _L)] = acc
                    return c2
                lax.fori_loop(0, nodes, inner, 0, unroll=4)
                return carry
            lax.fori_loop(0, _L, outer, 0)

        pltpu.sync_copy(hx_hbm.at[pl.ds(blk0 * idx_blk, bpw * idx_blk)],
                        idx_all)
        g_copy(0, rows0, gsem0).start()

        def half(b, rows, out_v, gsem, ssem, m):
            g_copy(b, rows, gsem).wait()

            @pl.when(m >= 1)
            def _():
                s_copy(b, out_v, ssem).wait()
            compute(rows, out_v)
            s_copy(b, out_v, ssem).start()

        def pair(m, carry):
            b0 = 2 * m
            g_copy(b0 + 1, rows1, gsem1).start()
            half(b0, rows0, out0, gsem0, ssem0, m)

            @pl.when(m < bpw // 2 - 1)
            def _():
                g_copy(b0 + 2, rows0, gsem0).start()
            half(b0 + 1, rows1, out1, gsem1, ssem1, m)
            return carry

        lax.fori_loop(0, bpw // 2, pair, 0)
        s_copy(bpw - 2, out0, ssem0).wait()
        s_copy(bpw - 1, out1, ssem1).wait()

        if n_tail:
            @pl.when(wid == 0)
            def _tail():
                tidx = n_tail * k
                pltpu.sync_copy(hx_hbm.at[pl.ds(n_main * k, tidx)],
                                idx_all.at[pl.ds(0, tidx)])
                iv = idx_all[pl.ds(0, _L)]
                idx_all[pl.ds(0, _L)] = jnp.where(lane < tidx, iv, 0)
                pltpu.async_copy(
                    x_hbm.at[idx_all.at[pl.ds(0, _L)]],
                    rows0.at[pl.ds(0, _L)], gsem0).wait()
                compute(rows0, out0, nodes=n_tail)
                pltpu.sync_copy(out0.at[pl.ds(0, n_tail)],
                                out_hbm.at[pl.ds(n_main, n_tail)])

    return body(x, hx)
